# baseline (speedup 1.0000x reference)
import functools

import jax
import jax.numpy as jnp
from jax import lax
from jax.experimental import pallas as pl
from jax.experimental.pallas import tpu as pltpu
from jax.experimental.pallas import tpu_sc as plsc

N_Z = 16
BATCH = 16384
N_GROUP = 1000000
NC = 2
NS = 16
NW = NC * NS
B_PER_W = BATCH // NW
CH = 128
N_CH = B_PER_W // CH
TILES = CH // N_Z
IDX_CHUNK = 128
N_ICH = B_PER_W // IDX_CHUNK


def _sc_body(x_hbm, idx_hbm, emb1_hbm, emb2_hbm, out_hbm,
             idx_v, m_v, ic_v, ac_v, b_v, x_v, o_v, sem_a, sem_b, sem_x):
    wid = lax.axis_index("s") * NC + lax.axis_index("c")
    base = wid * B_PER_W
    base_blk = base // 8

    pltpu.sync_copy(idx_hbm.at[pl.ds(base, B_PER_W)], idx_v)

    bcps = []
    for g in range(N_ICH):
        sl = pl.ds(g * IDX_CHUNK, IDX_CHUNK)
        bcps.append(pltpu.async_copy(
            emb2_hbm.at[idx_v.at[sl]], b_v.at[sl], sem_b))

    for g in range(B_PER_W // N_Z):
        sl = pl.ds(g * N_Z, N_Z)
        m_v[sl] = idx_v[sl] * N_Z

    lanes = lax.iota(jnp.int32, N_Z)
    xs = lanes % 8

    def step(c, _):
        buf = lax.bitwise_and(c, 1)

        @pl.when(c < N_CH)
        def _issue():
            pltpu.async_copy(
                x_hbm.at[pl.ds(base_blk + (CH // 8) * c, CH // 8)],
                x_v.at[buf], sem_x)
            for col in range(N_Z):
                for g in range(CH // N_Z):
                    sl = pl.ds(g * N_Z, N_Z)
                    ic_v[buf, col, sl] = m_v[pl.ds(c * CH + g * N_Z, N_Z)] + col
                pltpu.async_copy(
                    emb1_hbm.at[ic_v.at[buf, col]], ac_v.at[buf, col], sem_a)

        @pl.when(c > 0)
        def _drain_compute():
            p = c - 1
            pbuf = lax.bitwise_and(p, 1)
            pltpu.make_async_copy(
                x_hbm.at[pl.ds(0, CH // 8)], x_v.at[0], sem_x).wait()
            for col in range(N_Z):
                pltpu.make_async_copy(
                    emb1_hbm.at[pl.ds(0, CH)], ac_v.at[0, 0], sem_a).wait()
            bufv = jnp.full((N_Z,), pbuf, jnp.int32)
            for t in range(TILES):
                xjt = 2 * t + lanes // 8
                acc = b_v[pl.ds(p * CH + t * N_Z, N_Z)]
                for col in range(N_Z):
                    colv = jnp.full((N_Z,), col, jnp.int32)
                    xc = plsc.load_gather(x_v, [bufv, xjt, xs, colv])
                    ac = ac_v[pbuf, col, pl.ds(t * N_Z, N_Z)]
                    acc = acc + xc * ac
                o_v[pl.ds(p * CH + t * N_Z, N_Z)] = acc

        return 0

    for cp in bcps:
        cp.wait()
    lax.fori_loop(0, N_CH + 1, step, 0)
    pltpu.sync_copy(o_v, out_hbm.at[pl.ds(base, B_PER_W)])


@jax.jit
def _rand_effect(x3, idx, emb1_f, emb2_f):
    mesh = plsc.VectorSubcoreMesh(core_axis_name="c", subcore_axis_name="s")
    k = functools.partial(
        pl.kernel,
        out_type=jax.ShapeDtypeStruct((BATCH,), jnp.float32),
        mesh=mesh,
        compiler_params=pltpu.CompilerParams(needs_layout_passes=False),
        scratch_types=[
            pltpu.VMEM((B_PER_W,), jnp.int32),
            pltpu.VMEM((B_PER_W,), jnp.int32),
            pltpu.VMEM((2, N_Z, CH), jnp.int32),
            pltpu.VMEM((2, N_Z, CH), jnp.float32),
            pltpu.VMEM((B_PER_W,), jnp.float32),
            pltpu.VMEM((2, CH // 8, 8, N_Z), jnp.float32),
            pltpu.VMEM((B_PER_W,), jnp.float32),
            pltpu.SemaphoreType.DMA,
            pltpu.SemaphoreType.DMA,
            pltpu.SemaphoreType.DMA,
        ],
    )(_sc_body)
    return k(x3, idx, emb1_f, emb2_f)


def kernel(x, idx, emb1, emb2):
    x3 = x.reshape(BATCH // 8, 8, N_Z)
    emb1_f = emb1.reshape(-1)
    emb2_f = emb2.reshape(-1)
    out = _rand_effect(x3, idx.astype(jnp.int32), emb1_f, emb2_f)
    return out.reshape(BATCH, 1)


# speedup vs baseline: 2.5783x; 2.5783x over previous
import functools

import jax
import jax.numpy as jnp
from jax import lax
from jax.experimental import pallas as pl
from jax.experimental.pallas import tpu as pltpu
from jax.experimental.pallas import tpu_sc as plsc

N_Z = 16
BATCH = 16384
N_GROUP = 1000000
NC = 2
NS = 16
NW = NC * NS
B_PER_W = BATCH // NW
CH = 16
N_CH = B_PER_W // CH
IDX_CHUNK = 128
N_ICH = B_PER_W // IDX_CHUNK


def _sc_body(x_hbm, idx_hbm, emb1_hbm, emb2_hbm, out_hbm,
             idx_v, a_v, b_v, x_v, o_v, sem_a, sem_b, sem_x):
    wid = lax.axis_index("s") * NC + lax.axis_index("c")
    base = wid * B_PER_W
    base_blk = base // 8

    pltpu.sync_copy(idx_hbm.at[pl.ds(base, B_PER_W)], idx_v)

    bcps = []
    for g in range(N_ICH):
        sl = pl.ds(g * IDX_CHUNK, IDX_CHUNK)
        bcps.append(pltpu.async_copy(
            emb2_hbm.at[idx_v.at[sl]], b_v.at[sl], sem_b))

    lanes = lax.iota(jnp.int32, N_Z)
    xj = lanes // 8
    xs = lanes % 8

    def step(c, _):
        buf = lax.bitwise_and(c, 1)

        @pl.when(c < N_CH)
        def _issue():
            idx16 = idx_v[pl.ds(c * CH, CH)]
            blk16 = lax.shift_right_logical(idx16, 3)
            pltpu.async_copy(
                x_hbm.at[pl.ds(base_blk + 2 * c, 2)], x_v.at[buf], sem_x)
            for r in range(CH):
                blk = blk16[r]
                pltpu.async_copy(emb1_hbm.at[blk], a_v.at[buf, r], sem_a)

        @pl.when(c > 0)
        def _drain_compute():
            p = c - 1
            pbuf = lax.bitwise_and(p, 1)
            pltpu.make_async_copy(
                x_hbm.at[pl.ds(0, 2)], x_v.at[0], sem_x).wait()
            for r in range(CH):
                pltpu.make_async_copy(
                    emb1_hbm.at[0], a_v.at[0, r], sem_a).wait()
            idx16 = idx_v[pl.ds(p * CH, CH)]
            sub16 = lax.bitwise_and(idx16, 7)
            bufv = jnp.full((N_Z,), pbuf, jnp.int32)
            acc = b_v[pl.ds(p * CH, CH)]
            for col in range(N_Z):
                colv = jnp.full((N_Z,), col, jnp.int32)
                xc = plsc.load_gather(x_v, [bufv, xj, xs, colv])
                ac = plsc.load_gather(a_v, [bufv, lanes, sub16, colv])
                acc = acc + xc * ac
            o_v[pl.ds(p * CH, CH)] = acc

        return 0

    for cp in bcps:
        cp.wait()
    lax.fori_loop(0, N_CH + 1, step, 0)
    pltpu.sync_copy(o_v, out_hbm.at[pl.ds(base, B_PER_W)])


@jax.jit
def _rand_effect(x3, idx, emb1_3, emb2_f):
    mesh = plsc.VectorSubcoreMesh(core_axis_name="c", subcore_axis_name="s")
    k = functools.partial(
        pl.kernel,
        out_type=jax.ShapeDtypeStruct((BATCH,), jnp.float32),
        mesh=mesh,
        compiler_params=pltpu.CompilerParams(needs_layout_passes=False),
        scratch_types=[
            pltpu.VMEM((B_PER_W,), jnp.int32),
            pltpu.VMEM((2, CH, 8, N_Z), jnp.float32),
            pltpu.VMEM((B_PER_W,), jnp.float32),
            pltpu.VMEM((2, 2, 8, N_Z), jnp.float32),
            pltpu.VMEM((B_PER_W,), jnp.float32),
            pltpu.SemaphoreType.DMA,
            pltpu.SemaphoreType.DMA,
            pltpu.SemaphoreType.DMA,
        ],
    )(_sc_body)
    return k(x3, idx, emb1_3, emb2_f)


def kernel(x, idx, emb1, emb2):
    x3 = x.reshape(BATCH // 8, 8, N_Z)
    emb1_3 = emb1.reshape(N_GROUP // 8, 8, N_Z)
    emb2_f = emb2.reshape(-1)
    out = _rand_effect(x3, idx.astype(jnp.int32), emb1_3, emb2_f)
    return out.reshape(BATCH, 1)


# speedup vs baseline: 2.6193x; 1.0159x over previous
import functools

import jax
import jax.numpy as jnp
from jax import lax
from jax.experimental import pallas as pl
from jax.experimental.pallas import tpu as pltpu
from jax.experimental.pallas import tpu_sc as plsc

N_Z = 16
BATCH = 16384
N_GROUP = 1000000
NC = 2
NS = 16
NW = NC * NS
B_PER_W = BATCH // NW
CH = 32
N_CH = B_PER_W // CH
IDX_CHUNK = 128
N_ICH = B_PER_W // IDX_CHUNK


def _sc_body(x_hbm, idx_hbm, emb1_hbm, emb2_hbm, out_hbm,
             idx_v, a_v, b_v, x_v, o_v, sem_a, sem_b, sem_x):
    wid = lax.axis_index("s") * NC + lax.axis_index("c")
    base = wid * B_PER_W
    base_blk = base // 8

    pltpu.sync_copy(idx_hbm.at[pl.ds(base, B_PER_W)], idx_v)

    bcps = []
    for g in range(N_ICH):
        sl = pl.ds(g * IDX_CHUNK, IDX_CHUNK)
        bcps.append(pltpu.async_copy(
            emb2_hbm.at[idx_v.at[sl]], b_v.at[sl], sem_b))

    lanes = lax.iota(jnp.int32, N_Z)
    xj = lanes // 8
    xs = lanes % 8

    def step(c, _):
        buf = lax.bitwise_and(c, 1)

        @pl.when(c < N_CH)
        def _issue():
            blkh = []
            for h in range(2):
                idx16 = idx_v[pl.ds(c * CH + h * N_Z, N_Z)]
                blkh.append(lax.shift_right_logical(idx16, 3))
            pltpu.async_copy(
                x_hbm.at[pl.ds(base_blk + 4 * c, 4)], x_v.at[buf], sem_x)
            for r in range(CH):
                blk = blkh[r // N_Z][r % N_Z]
                pltpu.async_copy(emb1_hbm.at[blk], a_v.at[buf, r], sem_a)

        @pl.when(c > 0)
        def _drain_compute():
            p = c - 1
            pbuf = lax.bitwise_and(p, 1)
            pltpu.make_async_copy(
                x_hbm.at[pl.ds(0, 4)], x_v.at[0], sem_x).wait()
            for r in range(CH):
                pltpu.make_async_copy(
                    emb1_hbm.at[0], a_v.at[0, r], sem_a).wait()
            bufv = jnp.full((N_Z,), pbuf, jnp.int32)
            for h in range(2):
                idx16 = idx_v[pl.ds(p * CH + h * N_Z, N_Z)]
                sub16 = lax.bitwise_and(idx16, 7)
                jv = lanes + h * N_Z
                xjh = xj + 2 * h
                acc = b_v[pl.ds(p * CH + h * N_Z, N_Z)]
                for col in range(N_Z):
                    colv = jnp.full((N_Z,), col, jnp.int32)
                    xc = plsc.load_gather(x_v, [bufv, xjh, xs, colv])
                    ac = plsc.load_gather(a_v, [bufv, jv, sub16, colv])
                    acc = acc + xc * ac
                o_v[pl.ds(p * CH + h * N_Z, N_Z)] = acc

        return 0

    for cp in bcps:
        cp.wait()
    lax.fori_loop(0, N_CH + 1, step, 0)
    pltpu.sync_copy(o_v, out_hbm.at[pl.ds(base, B_PER_W)])


@jax.jit
def _rand_effect(x3, idx, emb1_3, emb2_f):
    mesh = plsc.VectorSubcoreMesh(core_axis_name="c", subcore_axis_name="s")
    k = functools.partial(
        pl.kernel,
        out_type=jax.ShapeDtypeStruct((BATCH,), jnp.float32),
        mesh=mesh,
        compiler_params=pltpu.CompilerParams(needs_layout_passes=False),
        scratch_types=[
            pltpu.VMEM((B_PER_W,), jnp.int32),
            pltpu.VMEM((2, CH, 8, N_Z), jnp.float32),
            pltpu.VMEM((B_PER_W,), jnp.float32),
            pltpu.VMEM((2, 4, 8, N_Z), jnp.float32),
            pltpu.VMEM((B_PER_W,), jnp.float32),
            pltpu.SemaphoreType.DMA,
            pltpu.SemaphoreType.DMA,
            pltpu.SemaphoreType.DMA,
        ],
    )(_sc_body)
    return k(x3, idx, emb1_3, emb2_f)


def kernel(x, idx, emb1, emb2):
    x3 = x.reshape(BATCH // 8, 8, N_Z)
    emb1_3 = emb1.reshape(N_GROUP // 8, 8, N_Z)
    emb2_f = emb2.reshape(-1)
    out = _rand_effect(x3, idx.astype(jnp.int32), emb1_3, emb2_f)
    return out.reshape(BATCH, 1)
